# baseline (speedup 1.0000x reference)
import functools

import numpy as np
import ml_dtypes

import jax
import jax.numpy as jnp
from jax import lax
from jax.experimental import pallas as pl
from jax.experimental.pallas import tpu as pltpu
from jax.experimental.pallas import tpu_sc as plsc


def _build_lut256() -> np.ndarray:
    b = np.arange(256, dtype=np.uint8).view(ml_dtypes.float8_e4m3fn)
    return b.astype(np.float32).view(np.int32)


_LUT256 = _build_lut256()

_NW = 32


def _sc_lookup(bsz: int, seq: int, v: int, h: int):
    b_per_w = bsz // _NW
    n_pair = b_per_w // 2
    mesh = plsc.VectorSubcoreMesh(core_axis_name="c", subcore_axis_name="s")

    @functools.partial(
        pl.kernel,
        out_type=jax.ShapeDtypeStruct((seq, bsz, h), jnp.bfloat16),
        mesh=mesh,
        scratch_types=[
            pltpu.VMEM((256,), jnp.int32),
            pltpu.VMEM((b_per_w, seq), jnp.int32),
            pltpu.VMEM((seq, b_per_w), jnp.int32),
            pltpu.VMEM((seq, b_per_w), jnp.int32),
        ] + [pltpu.VMEM((b_per_w, 128), jnp.int32)] * 2
          + [pltpu.VMEM((b_per_w,), jnp.int32)] * 2
          + [pltpu.VMEM((n_pair, 128), jnp.int32)] * 2
          + [pltpu.SemaphoreType.DMA] * 6,
        compiler_params=pltpu.CompilerParams(needs_layout_passes=False),
    )
    def k(idx_hbm, w_hbm, s_hbm, lut_hbm, out_hbm,
          lut_v, idx2d_v, idxT_v, idxq_v,
          in_0, in_1, sc_0, sc_1, out_0, out_1,
          sem_w0, sem_w1, sem_s0, sem_s1, sem_o0, sem_o1):
        in_b = (in_0, in_1)
        sc_b = (sc_0, sc_1)
        out_b = (out_0, out_1)
        sem_w = (sem_w0, sem_w1)
        sem_s = (sem_s0, sem_s1)
        sem_o = (sem_o0, sem_o1)

        w_line = w_hbm.bitcast(jnp.int32)
        o32 = out_hbm.bitcast(jnp.int32)

        wid = lax.axis_index("s") * 2 + lax.axis_index("c")
        pltpu.sync_copy(lut_hbm, lut_v)
        pltpu.sync_copy(idx_hbm.at[pl.ds(wid * b_per_w, b_per_w), :], idx2d_v)

        iota16 = lax.iota(jnp.int32, 16)

        def tr_body(l, carry):
            l16 = jnp.full((16,), l, jnp.int32)
            for m in range(8):
                col = plsc.load_gather(idx2d_v, [iota16 + 16 * m, l16])
                idxT_v[l, pl.ds(16 * m, 16)] = col
                idxq_v[l, pl.ds(16 * m, 16)] = lax.shift_right_logical(col, 2)
            return carry

        lax.fori_loop(0, seq, tr_body, 0)

        def issue(l, b):
            pltpu.async_copy(w_line.at[idxq_v.at[l]], in_b[b], sem_w[b])
            pltpu.async_copy(s_hbm.at[idxT_v.at[l]], sc_b[b], sem_s[b])

        def wait(l, b):
            pltpu.make_async_copy(w_line.at[idxq_v.at[l]], in_b[b],
                                  sem_w[b]).wait()
            pltpu.make_async_copy(s_hbm.at[idxT_v.at[l]], sc_b[b],
                                  sem_s[b]).wait()

        def owb(l, b):
            return pltpu.make_async_copy(
                out_b[b], o32.at[l, pl.ds(wid * n_pair, n_pair)], sem_o[b])

        def decode_chunk(l, b):
            in_v = in_b[b]
            sc_v = sc_b[b]
            out_v = out_b[b]
            l16 = jnp.full((16,), l, jnp.int32)

            @pl.when(l >= 2)
            def _():
                owb(l - 2, b).wait()

            @plsc.parallel_loop(0, n_pair, unroll=8)
            def pair_body(t):
                r0 = 2 * t
                r1 = 2 * t + 1
                raw0 = plsc.load_gather(idxT_v, [l16, jnp.full((16,), r0,
                                                               jnp.int32)])
                raw1 = plsc.load_gather(idxT_v, [l16, jnp.full((16,), r1,
                                                               jnp.int32)])
                sp0 = plsc.load_gather(sc_v, [jnp.full((16,), r0, jnp.int32)])
                sp1 = plsc.load_gather(sc_v, [jnp.full((16,), r1, jnp.int32)])
                sf0 = plsc.bitcast(sp0 << 16, jnp.float32)
                sf1 = plsc.bitcast(sp1 << 16, jnp.float32)
                sh0 = plsc.bitcast((raw0 & 3) * 8, jnp.uint32)
                sh1 = plsc.bitcast((raw1 & 3) * 8, jnp.uint32)
                for g in range(8):
                    w0 = plsc.bitcast(in_v[r0, pl.ds(16 * g, 16)], jnp.uint32)
                    w1 = plsc.bitcast(in_v[r1, pl.ds(16 * g, 16)], jnp.uint32)
                    b0 = plsc.bitcast((w0 >> sh0) & 0xFF, jnp.int32)
                    b1 = plsc.bitcast((w1 >> sh1) & 0xFF, jnp.int32)
                    f0 = plsc.bitcast(plsc.load_gather(lut_v, [b0]),
                                      jnp.float32) * sf0
                    f1 = plsc.bitcast(plsc.load_gather(lut_v, [b1]),
                                      jnp.float32) * sf1
                    pk = plsc.pack(f0, f1, format=plsc.PackFormat.INTERLEAVED)
                    out_v[t, pl.ds(16 * g, 16)] = plsc.bitcast(pk, jnp.int32)

            pltpu.async_copy(out_v, o32.at[l, pl.ds(wid * n_pair, n_pair)],
                             sem_o[b])

        issue(0, 0)

        def body2(ll, carry):
            for b in range(2):
                l = ll * 2 + b

                @pl.when(l + 1 < seq)
                def _():
                    issue(l + 1, 1 - b)

                wait(l, b)
                decode_chunk(l, b)
            return carry

        lax.fori_loop(0, seq // 2, body2, 0)
        owb(seq - 2, 0).wait()
        owb(seq - 1, 1).wait()

    return k


def kernel(indices, weight, scale):
    b, l = indices.shape
    v, h = weight.shape

    sbits = lax.bitcast_convert_type(scale.reshape(v), jnp.uint16).astype(jnp.uint32)
    s_dup = lax.bitcast_convert_type(sbits | (sbits << 16), jnp.int32)
    lut = jnp.asarray(_LUT256)

    out = _sc_lookup(b, l, v, h)(indices, weight, s_dup, lut)
    return out.transpose(1, 0, 2)


# speedup vs baseline: 1.1567x; 1.1567x over previous
import functools

import numpy as np
import ml_dtypes

import jax
import jax.numpy as jnp
from jax import lax
from jax.experimental import pallas as pl
from jax.experimental.pallas import tpu as pltpu
from jax.experimental.pallas import tpu_sc as plsc


def _build_lut256() -> np.ndarray:
    b = np.arange(256, dtype=np.uint8).view(ml_dtypes.float8_e4m3fn)
    return b.astype(np.float32).view(np.int32)


_LUT256 = _build_lut256()

_NW = 32


def _sc_lookup(bsz: int, seq: int, v: int, h: int):
    b_per_w = bsz // _NW
    n_pair = b_per_w // 2
    mesh = plsc.VectorSubcoreMesh(core_axis_name="c", subcore_axis_name="s")

    @functools.partial(
        pl.kernel,
        out_type=jax.ShapeDtypeStruct((seq, bsz, h), jnp.bfloat16),
        mesh=mesh,
        scratch_types=[
            pltpu.VMEM((256,), jnp.int32),
            pltpu.VMEM((b_per_w, seq), jnp.int32),
            pltpu.VMEM((seq, b_per_w), jnp.int32),
            pltpu.VMEM((seq, b_per_w), jnp.int32),
        ] + [pltpu.VMEM((b_per_w, 128), jnp.int32)] * 2
          + [pltpu.VMEM((b_per_w,), jnp.int32)] * 2
          + [pltpu.VMEM((n_pair, 128), jnp.int32)] * 2
          + [pltpu.SemaphoreType.DMA] * 6,
        compiler_params=pltpu.CompilerParams(needs_layout_passes=False),
    )
    def k(idx_hbm, w_hbm, s_hbm, lut_hbm, out_hbm,
          lut_v, idx2d_v, idxT_v, idxq_v,
          in_0, in_1, sc_0, sc_1, out_0, out_1,
          sem_w0, sem_w1, sem_s0, sem_s1, sem_o0, sem_o1):
        in_b = (in_0, in_1)
        sc_b = (sc_0, sc_1)
        out_b = (out_0, out_1)
        sem_w = (sem_w0, sem_w1)
        sem_s = (sem_s0, sem_s1)
        sem_o = (sem_o0, sem_o1)

        w_line = w_hbm.bitcast(jnp.int32)
        o32 = out_hbm.bitcast(jnp.int32)

        wid = lax.axis_index("s") * 2 + lax.axis_index("c")
        pltpu.sync_copy(lut_hbm, lut_v)
        pltpu.sync_copy(idx_hbm.at[pl.ds(wid * b_per_w, b_per_w), :], idx2d_v)

        iota16 = lax.iota(jnp.int32, 16)

        def tr_body(l, carry):
            l16 = jnp.full((16,), l, jnp.int32)
            for m in range(8):
                col = plsc.load_gather(idx2d_v, [iota16 + 16 * m, l16])
                idxT_v[l, pl.ds(16 * m, 16)] = col
                idxq_v[l, pl.ds(16 * m, 16)] = lax.shift_right_logical(col, 2)
            return carry

        def issue(l, b):
            pltpu.async_copy(w_line.at[idxq_v.at[l]], in_b[b], sem_w[b])
            pltpu.async_copy(s_hbm.at[idxT_v.at[l]], sc_b[b], sem_s[b])

        def wait(l, b):
            pltpu.make_async_copy(w_line.at[idxq_v.at[l]], in_b[b],
                                  sem_w[b]).wait()
            pltpu.make_async_copy(s_hbm.at[idxT_v.at[l]], sc_b[b],
                                  sem_s[b]).wait()

        def owb(l, b):
            return pltpu.make_async_copy(
                out_b[b], o32.at[l, pl.ds(wid * n_pair, n_pair)], sem_o[b])

        def decode_chunk(l, b):
            in_v = in_b[b]
            sc_v = sc_b[b]
            out_v = out_b[b]
            l16 = jnp.full((16,), l, jnp.int32)

            @pl.when(l >= 2)
            def _():
                owb(l - 2, b).wait()

            @plsc.parallel_loop(0, n_pair, unroll=4)
            def pair_body(t):
                r0 = 2 * t
                r1 = 2 * t + 1
                raw0 = plsc.load_gather(idxT_v, [l16, jnp.full((16,), r0,
                                                               jnp.int32)])
                raw1 = plsc.load_gather(idxT_v, [l16, jnp.full((16,), r1,
                                                               jnp.int32)])
                sp0 = plsc.load_gather(sc_v, [jnp.full((16,), r0, jnp.int32)])
                sp1 = plsc.load_gather(sc_v, [jnp.full((16,), r1, jnp.int32)])
                sf0 = plsc.bitcast(sp0 << 16, jnp.float32)
                sf1 = plsc.bitcast(sp1 << 16, jnp.float32)
                sh0 = plsc.bitcast((raw0 & 3) * 8, jnp.uint32)
                sh1 = plsc.bitcast((raw1 & 3) * 8, jnp.uint32)
                for g in range(8):
                    w0 = plsc.bitcast(in_v[r0, pl.ds(16 * g, 16)], jnp.uint32)
                    w1 = plsc.bitcast(in_v[r1, pl.ds(16 * g, 16)], jnp.uint32)
                    b0 = plsc.bitcast((w0 >> sh0) & 0xFF, jnp.int32)
                    b1 = plsc.bitcast((w1 >> sh1) & 0xFF, jnp.int32)
                    f0 = plsc.bitcast(plsc.load_gather(lut_v, [b0]),
                                      jnp.float32) * sf0
                    f1 = plsc.bitcast(plsc.load_gather(lut_v, [b1]),
                                      jnp.float32) * sf1
                    pk = plsc.pack(f0, f1, format=plsc.PackFormat.INTERLEAVED)
                    out_v[t, pl.ds(16 * g, 16)] = plsc.bitcast(pk, jnp.int32)

            pltpu.async_copy(out_v, o32.at[l, pl.ds(wid * n_pair, n_pair)],
                             sem_o[b])

        tr_body(0, 0)
        issue(0, 0)
        lax.fori_loop(1, seq, tr_body, 0)

        def body2(ll, carry):
            for b in range(2):
                l = ll * 2 + b

                @pl.when(l + 1 < seq)
                def _():
                    issue(l + 1, 1 - b)

                wait(l, b)
                decode_chunk(l, b)
            return carry

        lax.fori_loop(0, seq // 2, body2, 0)
        owb(seq - 2, 0).wait()
        owb(seq - 1, 1).wait()

    return k


def kernel(indices, weight, scale):
    b, l = indices.shape
    v, h = weight.shape

    sbits = lax.bitcast_convert_type(scale.reshape(v), jnp.uint16).astype(jnp.uint32)
    s_dup = lax.bitcast_convert_type(sbits | (sbits << 16), jnp.int32)
    lut = jnp.asarray(_LUT256)

    out = _sc_lookup(b, l, v, h)(indices, weight, s_dup, lut)
    return out.transpose(1, 0, 2)
